# baseline (speedup 1.0000x reference)
import functools

import jax
import jax.numpy as jnp
from jax.experimental import pallas as pl
from jax.experimental.pallas import tpu as pltpu


def _linear_sigmoid_pipeline(x_hbm, w_ref, b_ref, o_hbm, xbuf, obuf, rsem,
                             wsem, rsem0, wseml, *, nslabs, slab, chunk,
                             nbuf):
    nchunk = slab // chunk
    wb = w_ref[...].astype(jnp.bfloat16)
    b = b_ref[...]
    last = nslabs - 1

    def read(s):
        slot = s % nbuf
        return pltpu.make_async_copy(
            x_hbm.at[pl.ds(s * slab, slab), :], xbuf.at[slot], rsem.at[slot])

    def write(s):
        slot = s % nbuf
        return pltpu.make_async_copy(
            obuf.at[slot], o_hbm.at[pl.ds(s * slab, slab), :], wsem.at[slot])

    head = chunk
    read0a = pltpu.make_async_copy(
        x_hbm.at[pl.ds(0, head), :], xbuf.at[0, pl.ds(0, head), :],
        rsem0.at[0])
    read0b = pltpu.make_async_copy(
        x_hbm.at[pl.ds(head, slab - head), :],
        xbuf.at[0, pl.ds(head, slab - head), :], rsem0.at[1])

    def write_last_chunk(r):
        slot = last % nbuf
        return pltpu.make_async_copy(
            obuf.at[slot, pl.ds(r * chunk, chunk), :],
            o_hbm.at[pl.ds(last * slab + r * chunk, chunk), :], wseml.at[r])

    read0a.start()
    read0b.start()
    for s in range(1, min(nbuf, nslabs)):
        read(s).start()

    started = set()

    def start_write(s):
        if s not in started:
            started.add(s)
            write(s).start()

    for s in range(nslabs):
        slot = s % nbuf
        if s == 0:
            read0a.wait()
        else:
            read(s).wait()
        if s >= nbuf:
            start_write(s - nbuf)
            write(s - nbuf).wait()
        for r in range(nchunk):
            if s == 0 and r == 1:
                read0b.wait()
            xs = xbuf[slot, pl.ds(r * chunk, chunk), :].astype(jnp.bfloat16)
            acc = jnp.dot(xs, wb, preferred_element_type=jnp.float32)
            obuf[slot, pl.ds(r * chunk, chunk), :] = jax.nn.sigmoid(acc + b)
            if s == last:
                write_last_chunk(r).start()
        if s != last and (s % 2 == 1 or s == last - 1):
            if s > 0:
                start_write(s - 1)
            start_write(s)
        if s + nbuf < nslabs:
            read(s + nbuf).start()

    for s in range(max(0, nslabs - nbuf), nslabs):
        if s == last:
            for r in range(nchunk):
                write_last_chunk(r).wait()
        else:
            start_write(s)
            write(s).wait()


@jax.jit
def kernel(x, w_t, b2d):
    x = x.astype(jnp.float32)
    M, K = x.shape
    K2, N = w_t.shape
    assert K == K2 and b2d.shape == (1, N)

    w_t = w_t.astype(jnp.float32)
    b2d = b2d.astype(jnp.float32)

    slab = 1024
    nbuf = 5
    while M % slab != 0 and slab > 8:
        slab //= 2
    m_pad = M
    if M % slab != 0:
        m_pad = ((M + slab - 1) // slab) * slab
        x = jnp.pad(x, ((0, m_pad - M), (0, 0)))
    nslabs = m_pad // slab

    chunk = 256
    while slab % chunk != 0 and chunk > 8:
        chunk //= 2

    out = pl.pallas_call(
        functools.partial(_linear_sigmoid_pipeline,
                          nslabs=nslabs, slab=slab, chunk=chunk, nbuf=nbuf),
        out_shape=jax.ShapeDtypeStruct((m_pad, N), jnp.float32),
        in_specs=[
            pl.BlockSpec(memory_space=pl.ANY),
            pl.BlockSpec((K, N), lambda: (0, 0)),
            pl.BlockSpec((1, N), lambda: (0, 0)),
        ],
        out_specs=pl.BlockSpec(memory_space=pl.ANY),
        scratch_shapes=[
            pltpu.VMEM((nbuf, slab, K), jnp.float32),
            pltpu.VMEM((nbuf, slab, N), jnp.float32),
            pltpu.SemaphoreType.DMA((nbuf,)),
            pltpu.SemaphoreType.DMA((nbuf,)),
            pltpu.SemaphoreType.DMA((2,)),
            pltpu.SemaphoreType.DMA((slab // chunk,)),
        ],
    )(x, w_t, b2d)

    if m_pad != M:
        out = out[:M]
    return out


# speedup vs baseline: 1.0177x; 1.0177x over previous
import functools

import jax
import jax.numpy as jnp
from jax.experimental import pallas as pl
from jax.experimental.pallas import tpu as pltpu


def _linear_sigmoid_pipeline(x_hbm, w_ref, b_ref, o_hbm, xbuf, obuf, rsem,
                             wsem, rsem0, wseml, *, nslabs, slab, chunk,
                             nbuf):
    nchunk = slab // chunk
    wb = w_ref[...].astype(jnp.bfloat16)
    b = b_ref[...]
    last = nslabs - 1

    def read(s):
        slot = s % nbuf
        return pltpu.make_async_copy(
            x_hbm.at[pl.ds(s * slab, slab), :], xbuf.at[slot], rsem.at[slot])

    def write(s):
        slot = s % nbuf
        return pltpu.make_async_copy(
            obuf.at[slot], o_hbm.at[pl.ds(s * slab, slab), :], wsem.at[slot])

    head = chunk
    read0a = pltpu.make_async_copy(
        x_hbm.at[pl.ds(0, head), :], xbuf.at[0, pl.ds(0, head), :],
        rsem0.at[0])
    read0b = pltpu.make_async_copy(
        x_hbm.at[pl.ds(head, slab - head), :],
        xbuf.at[0, pl.ds(head, slab - head), :], rsem0.at[1])

    def write_last_chunk(r):
        slot = last % nbuf
        return pltpu.make_async_copy(
            obuf.at[slot, pl.ds(r * chunk, chunk), :],
            o_hbm.at[pl.ds(last * slab + r * chunk, chunk), :], wseml.at[r])

    read0a.start()
    read0b.start()
    for s in range(1, min(nbuf, nslabs)):
        read(s).start()

    started = set()

    def start_write(s):
        if s not in started:
            started.add(s)
            write(s).start()

    for s in range(nslabs):
        slot = s % nbuf
        if s == 0:
            read0a.wait()
        else:
            read(s).wait()
        if s >= nbuf:
            start_write(s - nbuf)
            write(s - nbuf).wait()
        for r in range(nchunk):
            if s == 0 and r == 1:
                read0b.wait()
            xs = xbuf[slot, pl.ds(r * chunk, chunk), :].astype(jnp.bfloat16)
            acc = jnp.dot(xs, wb, preferred_element_type=jnp.float32)
            obuf[slot, pl.ds(r * chunk, chunk), :] = jax.nn.sigmoid(acc + b)
            if s == last:
                write_last_chunk(r).start()
        if s != last and (s % 2 == 1 or s == last - 1):
            if s > 0:
                start_write(s - 1)
            start_write(s)
        if s + nbuf < nslabs:
            read(s + nbuf).start()

    for s in range(max(0, nslabs - nbuf), nslabs):
        if s == last:
            for r in range(nchunk):
                write_last_chunk(r).wait()
        else:
            start_write(s)
            write(s).wait()


@jax.jit
def kernel(x, w_t, b2d):
    x = x.astype(jnp.float32)
    M, K = x.shape
    K2, N = w_t.shape
    assert K == K2 and b2d.shape == (1, N)

    w_t = w_t.astype(jnp.float32)
    b2d = b2d.astype(jnp.float32)

    slab = 1024
    nbuf = 4
    while M % slab != 0 and slab > 8:
        slab //= 2
    m_pad = M
    if M % slab != 0:
        m_pad = ((M + slab - 1) // slab) * slab
        x = jnp.pad(x, ((0, m_pad - M), (0, 0)))
    nslabs = m_pad // slab

    chunk = 256
    while slab % chunk != 0 and chunk > 8:
        chunk //= 2

    out = pl.pallas_call(
        functools.partial(_linear_sigmoid_pipeline,
                          nslabs=nslabs, slab=slab, chunk=chunk, nbuf=nbuf),
        out_shape=jax.ShapeDtypeStruct((m_pad, N), jnp.float32),
        in_specs=[
            pl.BlockSpec(memory_space=pl.ANY),
            pl.BlockSpec((K, N), lambda: (0, 0)),
            pl.BlockSpec((1, N), lambda: (0, 0)),
        ],
        out_specs=pl.BlockSpec(memory_space=pl.ANY),
        scratch_shapes=[
            pltpu.VMEM((nbuf, slab, K), jnp.float32),
            pltpu.VMEM((nbuf, slab, N), jnp.float32),
            pltpu.SemaphoreType.DMA((nbuf,)),
            pltpu.SemaphoreType.DMA((nbuf,)),
            pltpu.SemaphoreType.DMA((2,)),
            pltpu.SemaphoreType.DMA((slab // chunk,)),
        ],
    )(x, w_t, b2d)

    if m_pad != M:
        out = out[:M]
    return out
